# Initial kernel scaffold; baseline (speedup 1.0000x reference)
import jax
import jax.numpy as jnp
from jax.experimental import pallas as pl


def kernel(x_user, x_item, edge_index_rates, edge_index_rated_by, W1_rates_self, W1_rates_neigh, W1_rb_self, W1_rb_neigh, W2_rates_self, W2_rates_neigh, W2_rb_self, W2_rb_neigh, b1_rates, b1_rb, b2_rates, b2_rb):
    raise NotImplementedError("write your pallas kernel here")


# speedup vs baseline: 10.7471x; 10.7471x over previous
import functools

import jax
import jax.numpy as jnp
from jax import lax
from jax.experimental import pallas as pl
from jax.experimental.pallas import tpu as pltpu
from jax.experimental.pallas import tpu_sc as plsc

N = 100000
D = 32
H = 16
E = 1600000
NS = 16
RPT = 784
ROWS = RPT * NS
EPAD = ROWS * 128
RB = 8
NBLK = RPT // RB
NACC = 100096
ZSTRIPE = NACC // NS
OTAIL = N - 15 * ZSTRIPE
ROWBLK = 2000
GRID = N // ROWBLK

_mesh = plsc.VectorSubcoreMesh(core_axis_name="c", subcore_axis_name="s")


def _copy_out(s, accum, out):
    @pl.when(s < NS - 1)
    def _():
        pltpu.sync_copy(accum.at[pl.ds(s * ZSTRIPE, ZSTRIPE)],
                        out.at[pl.ds(s * ZSTRIPE, ZSTRIPE)])

    @pl.when(s == NS - 1)
    def _():
        pltpu.sync_copy(accum.at[pl.ds((NS - 1) * ZSTRIPE, OTAIL)],
                        out.at[pl.ds((NS - 1) * ZSTRIPE, OTAIL)])


@functools.partial(
    pl.kernel,
    out_type=[jax.ShapeDtypeStruct((N, H), jnp.float32)] * 4,
    mesh=_mesh,
    compiler_params=pltpu.CompilerParams(use_tc_tiling_on_sc=False),
    scratch_types=[
        pltpu.VMEM_SHARED((NACC, H), jnp.float32),
        pltpu.VMEM((RB, 128), jnp.int32),
        pltpu.VMEM((RB, 128), jnp.int32),
        pltpu.VMEM((RB * 128, H), jnp.float32),
        pltpu.SemaphoreType.DMA,
    ],
)
def _sc_agg(tA_r, tB_r, tA_b, tB_b, srcR, dstR, srcB, dstB, zeros,
            oAi, oBi, oAu, oBu, accum, sidx, didx, rows, sem):
    c = lax.axis_index("c")
    s = lax.axis_index("s")

    def do_rel(table, src2, dst2, out):
        pltpu.sync_copy(zeros, accum.at[pl.ds(s * ZSTRIPE, ZSTRIPE)])
        plsc.subcore_barrier()
        base = s * RPT

        def blk(b, carry):
            r0 = base + b * RB
            pltpu.sync_copy(src2.at[pl.ds(r0, RB)], sidx)
            pltpu.sync_copy(dst2.at[pl.ds(r0, RB)], didx)
            cps = [
                pltpu.async_copy(table.at[sidx.at[j]],
                                 rows.at[pl.ds(j * 128, 128)], sem)
                for j in range(RB)
            ]
            for cp in cps:
                cp.wait()
            for j in range(RB):
                pltpu.sync_copy(rows.at[pl.ds(j * 128, 128)],
                                accum.at[didx.at[j]], add=True)
            return carry

        lax.fori_loop(0, NBLK, blk, 0)
        plsc.subcore_barrier()
        _copy_out(s, accum, out)
        plsc.subcore_barrier()

    @pl.when(c == 0)
    def _():
        do_rel(tA_r, srcR, dstR, oAi)
        do_rel(tA_b, srcB, dstB, oAu)

    @pl.when(c == 1)
    def _():
        do_rel(tB_r, srcR, dstR, oBi)
        do_rel(tB_b, srcB, dstB, oBu)


@functools.partial(
    pl.kernel,
    out_type=[jax.ShapeDtypeStruct((N, H), jnp.float32)] * 2,
    mesh=_mesh,
    compiler_params=pltpu.CompilerParams(use_tc_tiling_on_sc=False),
    scratch_types=[
        pltpu.VMEM_SHARED((NACC, H), jnp.float32),
        pltpu.VMEM((RB, 128), jnp.int32),
        pltpu.VMEM((128, H), jnp.float32),
    ],
)
def _sc_deg(dstR, dstB, ones, zeros, oI, oU, accum, didx, ones_v):
    c = lax.axis_index("c")
    s = lax.axis_index("s")
    pltpu.sync_copy(ones, ones_v)

    def do_rel(dst2, out):
        pltpu.sync_copy(zeros, accum.at[pl.ds(s * ZSTRIPE, ZSTRIPE)])
        plsc.subcore_barrier()
        base = s * RPT

        def blk(b, carry):
            r0 = base + b * RB
            pltpu.sync_copy(dst2.at[pl.ds(r0, RB)], didx)
            for j in range(RB):
                pltpu.sync_copy(ones_v, accum.at[didx.at[j]], add=True)
            return carry

        lax.fori_loop(0, NBLK, blk, 0)
        plsc.subcore_barrier()
        _copy_out(s, accum, out)

    @pl.when(c == 0)
    def _():
        do_rel(dstR, oI)

    @pl.when(c == 1)
    def _():
        do_rel(dstB, oU)


def _rows(w):
    return pl.BlockSpec((ROWBLK, w), lambda i: (i, 0))


_W = pl.BlockSpec((D, D), lambda i: (0, 0))
_B = pl.BlockSpec((1, D), lambda i: (0, 0))
_f32 = functools.partial(jnp.dot, preferred_element_type=jnp.float32)


def _tcA_body(xu, xi, wr, wb, yAr, yBr, yAb, yBb):
    yu = _f32(xu[...], wr[...])
    yi = _f32(xi[...], wb[...])
    yAr[...] = yu[:, :H]
    yBr[...] = yu[:, H:]
    yAb[...] = yi[:, :H]
    yBb[...] = yi[:, H:]


_tcA = pl.pallas_call(
    _tcA_body,
    grid=(GRID,),
    in_specs=[_rows(D), _rows(D), _W, _W],
    out_specs=[_rows(H)] * 4,
    out_shape=[jax.ShapeDtypeStruct((N, H), jnp.float32)] * 4,
)


def _tcB_body(xu, aAu, aBu, dgu, wsu, bu, wnr2,
              xi, aAi, aBi, dgi, wsi, bi, wnb2,
              hu, t2Ar, t2Br, hi, t2Ab, t2Bb):
    invu = 1.0 / jnp.maximum(dgu[...], 1.0)
    mu = jnp.concatenate([aAu[...] * invu, aBu[...] * invu], axis=1)
    h_u = jnp.maximum(_f32(xu[...], wsu[...]) + mu + bu[...], 0.0)
    hu[...] = h_u
    y2r = _f32(h_u, wnr2[...])
    t2Ar[...] = y2r[:, :H]
    t2Br[...] = y2r[:, H:]
    invi = 1.0 / jnp.maximum(dgi[...], 1.0)
    mi = jnp.concatenate([aAi[...] * invi, aBi[...] * invi], axis=1)
    h_i = jnp.maximum(_f32(xi[...], wsi[...]) + mi + bi[...], 0.0)
    hi[...] = h_i
    y2b = _f32(h_i, wnb2[...])
    t2Ab[...] = y2b[:, :H]
    t2Bb[...] = y2b[:, H:]


_tcB = pl.pallas_call(
    _tcB_body,
    grid=(GRID,),
    in_specs=[_rows(D), _rows(H), _rows(H), _rows(H), _W, _B, _W,
              _rows(D), _rows(H), _rows(H), _rows(H), _W, _B, _W],
    out_specs=[_rows(D), _rows(H), _rows(H), _rows(D), _rows(H), _rows(H)],
    out_shape=[jax.ShapeDtypeStruct((N, D), jnp.float32),
               jax.ShapeDtypeStruct((N, H), jnp.float32),
               jax.ShapeDtypeStruct((N, H), jnp.float32),
               jax.ShapeDtypeStruct((N, D), jnp.float32),
               jax.ShapeDtypeStruct((N, H), jnp.float32),
               jax.ShapeDtypeStruct((N, H), jnp.float32)],
)


def _tcC_body(hu, aAu, aBu, dgu, wsu, bu,
              hi, aAi, aBi, dgi, wsi, bi,
              ou, oi):
    invu = 1.0 / jnp.maximum(dgu[...], 1.0)
    mu = jnp.concatenate([aAu[...] * invu, aBu[...] * invu], axis=1)
    ou[...] = _f32(hu[...], wsu[...]) + mu + bu[...]
    invi = 1.0 / jnp.maximum(dgi[...], 1.0)
    mi = jnp.concatenate([aAi[...] * invi, aBi[...] * invi], axis=1)
    oi[...] = _f32(hi[...], wsi[...]) + mi + bi[...]


_tcC = pl.pallas_call(
    _tcC_body,
    grid=(GRID,),
    in_specs=[_rows(D), _rows(H), _rows(H), _rows(H), _W, _B,
              _rows(D), _rows(H), _rows(H), _rows(H), _W, _B],
    out_specs=[_rows(D), _rows(D)],
    out_shape=[jax.ShapeDtypeStruct((N, D), jnp.float32),
               jax.ShapeDtypeStruct((N, D), jnp.float32)],
)


def _pad_edges(src, dst):
    padn = EPAD - E
    pad_src = (jnp.arange(padn, dtype=jnp.int32) * 97) % N
    pad_dst = N + (jnp.arange(padn, dtype=jnp.int32) % (NACC - N))
    src2 = jnp.concatenate([src.astype(jnp.int32), pad_src]).reshape(ROWS, 128)
    dst2 = jnp.concatenate([dst.astype(jnp.int32), pad_dst]).reshape(ROWS, 128)
    return src2, dst2


def kernel(x_user, x_item, edge_index_rates, edge_index_rated_by,
           W1_rates_self, W1_rates_neigh, W1_rb_self, W1_rb_neigh,
           W2_rates_self, W2_rates_neigh, W2_rb_self, W2_rb_neigh,
           b1_rates, b1_rb, b2_rates, b2_rb):
    srcR2, dstR2 = _pad_edges(edge_index_rates[0], edge_index_rates[1])
    srcB2, dstB2 = _pad_edges(edge_index_rated_by[0], edge_index_rated_by[1])
    zeros = jnp.zeros((ZSTRIPE, H), jnp.float32)
    ones = jnp.ones((128, H), jnp.float32)

    degI, degU = _sc_deg(dstR2, dstB2, ones, zeros)
    yAr, yBr, yAb, yBb = _tcA(x_user, x_item, W1_rates_neigh, W1_rb_neigh)
    aAi, aBi, aAu, aBu = _sc_agg(yAr, yBr, yAb, yBb,
                                 srcR2, dstR2, srcB2, dstB2, zeros)
    hu, t2Ar, t2Br, hi, t2Ab, t2Bb = _tcB(
        x_user, aAu, aBu, degU, W1_rb_self, b1_rb.reshape(1, D), W2_rates_neigh,
        x_item, aAi, aBi, degI, W1_rates_self, b1_rates.reshape(1, D), W2_rb_neigh)
    a2Ai, a2Bi, a2Au, a2Bu = _sc_agg(t2Ar, t2Br, t2Ab, t2Bb,
                                     srcR2, dstR2, srcB2, dstB2, zeros)
    hu2, hi2 = _tcC(hu, a2Au, a2Bu, degU, W2_rb_self, b2_rb.reshape(1, D),
                    hi, a2Ai, a2Bi, degI, W2_rates_self, b2_rates.reshape(1, D))
    return (hu2, hi2)


# speedup vs baseline: 12.2767x; 1.1423x over previous
import functools

import jax
import jax.numpy as jnp
from jax import lax
from jax.experimental import pallas as pl
from jax.experimental.pallas import tpu as pltpu
from jax.experimental.pallas import tpu_sc as plsc

N = 100000
D = 32
H = 16
E = 1600000
NS = 16
RPT = 784
ROWS = RPT * NS
EPAD = ROWS * 128
RB = 8
NBLK = RPT // RB
NACC = 100096
ZSTRIPE = NACC // NS
OTAIL = N - 15 * ZSTRIPE
ROWBLK = 2000
GRID = N // ROWBLK

_mesh = plsc.VectorSubcoreMesh(core_axis_name="c", subcore_axis_name="s")


def _copy_out(s, accum, out):
    @pl.when(s < NS - 1)
    def _():
        pltpu.sync_copy(accum.at[pl.ds(s * ZSTRIPE, ZSTRIPE)],
                        out.at[pl.ds(s * ZSTRIPE, ZSTRIPE)])

    @pl.when(s == NS - 1)
    def _():
        pltpu.sync_copy(accum.at[pl.ds((NS - 1) * ZSTRIPE, OTAIL)],
                        out.at[pl.ds((NS - 1) * ZSTRIPE, OTAIL)])


@functools.partial(
    pl.kernel,
    out_type=[jax.ShapeDtypeStruct((N, H), jnp.float32)] * 4,
    mesh=_mesh,
    compiler_params=pltpu.CompilerParams(use_tc_tiling_on_sc=False),
    scratch_types=[
        pltpu.VMEM_SHARED((NACC, H), jnp.float32),
        pltpu.VMEM((RB, 128), jnp.int32),
        pltpu.VMEM((RB, 128), jnp.int32),
        pltpu.VMEM((RB * 128, H), jnp.float32),
        pltpu.SemaphoreType.DMA,
    ],
)
def _sc_agg(tA_r, tB_r, tA_b, tB_b, srcR, dstR, srcB, dstB, zeros,
            oAi, oBi, oAu, oBu, accum, sidx, didx, rows, sem):
    c = lax.axis_index("c")
    s = lax.axis_index("s")

    def do_rel(table, src2, dst2, out):
        pltpu.sync_copy(zeros, accum.at[pl.ds(s * ZSTRIPE, ZSTRIPE)])
        plsc.subcore_barrier()
        base = s * RPT

        def blk(b, carry):
            r0 = base + b * RB
            ics = [pltpu.async_copy(src2.at[pl.ds(r0, RB)], sidx, sem),
                   pltpu.async_copy(dst2.at[pl.ds(r0, RB)], didx, sem)]
            for cp in ics:
                cp.wait()
            gcs = [
                pltpu.async_copy(table.at[sidx.at[j]],
                                 rows.at[pl.ds(j * 128, 128)], sem)
                for j in range(RB)
            ]
            for cp in gcs:
                cp.wait()
            scs = [
                pltpu.async_copy(rows.at[pl.ds(j * 128, 128)],
                                 accum.at[didx.at[j]], sem, add=True)
                for j in range(RB)
            ]
            for cp in scs:
                cp.wait()
            return carry

        lax.fori_loop(0, NBLK, blk, 0)
        plsc.subcore_barrier()
        _copy_out(s, accum, out)
        plsc.subcore_barrier()

    @pl.when(c == 0)
    def _():
        do_rel(tA_r, srcR, dstR, oAi)
        do_rel(tA_b, srcB, dstB, oAu)

    @pl.when(c == 1)
    def _():
        do_rel(tB_r, srcR, dstR, oBi)
        do_rel(tB_b, srcB, dstB, oBu)


@functools.partial(
    pl.kernel,
    out_type=[jax.ShapeDtypeStruct((N, H), jnp.float32)] * 2,
    mesh=_mesh,
    compiler_params=pltpu.CompilerParams(use_tc_tiling_on_sc=False),
    scratch_types=[
        pltpu.VMEM_SHARED((NACC, H), jnp.float32),
        pltpu.VMEM((RB, 128), jnp.int32),
        pltpu.VMEM((128, H), jnp.float32),
        pltpu.SemaphoreType.DMA,
    ],
)
def _sc_deg(dstR, dstB, ones, zeros, oI, oU, accum, didx, ones_v, sem):
    c = lax.axis_index("c")
    s = lax.axis_index("s")
    pltpu.sync_copy(ones, ones_v)

    def do_rel(dst2, out):
        pltpu.sync_copy(zeros, accum.at[pl.ds(s * ZSTRIPE, ZSTRIPE)])
        plsc.subcore_barrier()
        base = s * RPT

        def blk(b, carry):
            r0 = base + b * RB
            pltpu.sync_copy(dst2.at[pl.ds(r0, RB)], didx)
            scs = [pltpu.async_copy(ones_v, accum.at[didx.at[j]], sem,
                                    add=True)
                   for j in range(RB)]
            for cp in scs:
                cp.wait()
            return carry

        lax.fori_loop(0, NBLK, blk, 0)
        plsc.subcore_barrier()
        _copy_out(s, accum, out)

    @pl.when(c == 0)
    def _():
        do_rel(dstR, oI)

    @pl.when(c == 1)
    def _():
        do_rel(dstB, oU)


def _rows(w):
    return pl.BlockSpec((ROWBLK, w), lambda i: (i, 0))


_W = pl.BlockSpec((D, D), lambda i: (0, 0))
_B = pl.BlockSpec((1, D), lambda i: (0, 0))
_f32 = functools.partial(jnp.dot, preferred_element_type=jnp.float32)


def _tcA_body(xu, xi, wr, wb, yAr, yBr, yAb, yBb):
    yu = _f32(xu[...], wr[...])
    yi = _f32(xi[...], wb[...])
    yAr[...] = yu[:, :H]
    yBr[...] = yu[:, H:]
    yAb[...] = yi[:, :H]
    yBb[...] = yi[:, H:]


_tcA = pl.pallas_call(
    _tcA_body,
    grid=(GRID,),
    in_specs=[_rows(D), _rows(D), _W, _W],
    out_specs=[_rows(H)] * 4,
    out_shape=[jax.ShapeDtypeStruct((N, H), jnp.float32)] * 4,
)


def _tcB_body(xu, aAu, aBu, dgu, wsu, bu, wnr2,
              xi, aAi, aBi, dgi, wsi, bi, wnb2,
              hu, t2Ar, t2Br, hi, t2Ab, t2Bb):
    invu = 1.0 / jnp.maximum(dgu[...], 1.0)
    mu = jnp.concatenate([aAu[...] * invu, aBu[...] * invu], axis=1)
    h_u = jnp.maximum(_f32(xu[...], wsu[...]) + mu + bu[...], 0.0)
    hu[...] = h_u
    y2r = _f32(h_u, wnr2[...])
    t2Ar[...] = y2r[:, :H]
    t2Br[...] = y2r[:, H:]
    invi = 1.0 / jnp.maximum(dgi[...], 1.0)
    mi = jnp.concatenate([aAi[...] * invi, aBi[...] * invi], axis=1)
    h_i = jnp.maximum(_f32(xi[...], wsi[...]) + mi + bi[...], 0.0)
    hi[...] = h_i
    y2b = _f32(h_i, wnb2[...])
    t2Ab[...] = y2b[:, :H]
    t2Bb[...] = y2b[:, H:]


_tcB = pl.pallas_call(
    _tcB_body,
    grid=(GRID,),
    in_specs=[_rows(D), _rows(H), _rows(H), _rows(H), _W, _B, _W,
              _rows(D), _rows(H), _rows(H), _rows(H), _W, _B, _W],
    out_specs=[_rows(D), _rows(H), _rows(H), _rows(D), _rows(H), _rows(H)],
    out_shape=[jax.ShapeDtypeStruct((N, D), jnp.float32),
               jax.ShapeDtypeStruct((N, H), jnp.float32),
               jax.ShapeDtypeStruct((N, H), jnp.float32),
               jax.ShapeDtypeStruct((N, D), jnp.float32),
               jax.ShapeDtypeStruct((N, H), jnp.float32),
               jax.ShapeDtypeStruct((N, H), jnp.float32)],
)


def _tcC_body(hu, aAu, aBu, dgu, wsu, bu,
              hi, aAi, aBi, dgi, wsi, bi,
              ou, oi):
    invu = 1.0 / jnp.maximum(dgu[...], 1.0)
    mu = jnp.concatenate([aAu[...] * invu, aBu[...] * invu], axis=1)
    ou[...] = _f32(hu[...], wsu[...]) + mu + bu[...]
    invi = 1.0 / jnp.maximum(dgi[...], 1.0)
    mi = jnp.concatenate([aAi[...] * invi, aBi[...] * invi], axis=1)
    oi[...] = _f32(hi[...], wsi[...]) + mi + bi[...]


_tcC = pl.pallas_call(
    _tcC_body,
    grid=(GRID,),
    in_specs=[_rows(D), _rows(H), _rows(H), _rows(H), _W, _B,
              _rows(D), _rows(H), _rows(H), _rows(H), _W, _B],
    out_specs=[_rows(D), _rows(D)],
    out_shape=[jax.ShapeDtypeStruct((N, D), jnp.float32),
               jax.ShapeDtypeStruct((N, D), jnp.float32)],
)


def _pad_edges(src, dst):
    padn = EPAD - E
    pad_src = (jnp.arange(padn, dtype=jnp.int32) * 97) % N
    pad_dst = N + (jnp.arange(padn, dtype=jnp.int32) % (NACC - N))
    src2 = jnp.concatenate([src.astype(jnp.int32), pad_src]).reshape(ROWS, 128)
    dst2 = jnp.concatenate([dst.astype(jnp.int32), pad_dst]).reshape(ROWS, 128)
    return src2, dst2


def kernel(x_user, x_item, edge_index_rates, edge_index_rated_by,
           W1_rates_self, W1_rates_neigh, W1_rb_self, W1_rb_neigh,
           W2_rates_self, W2_rates_neigh, W2_rb_self, W2_rb_neigh,
           b1_rates, b1_rb, b2_rates, b2_rb):
    srcR2, dstR2 = _pad_edges(edge_index_rates[0], edge_index_rates[1])
    srcB2, dstB2 = _pad_edges(edge_index_rated_by[0], edge_index_rated_by[1])
    zeros = jnp.zeros((ZSTRIPE, H), jnp.float32)
    ones = jnp.ones((128, H), jnp.float32)

    degI, degU = _sc_deg(dstR2, dstB2, ones, zeros)
    yAr, yBr, yAb, yBb = _tcA(x_user, x_item, W1_rates_neigh, W1_rb_neigh)
    aAi, aBi, aAu, aBu = _sc_agg(yAr, yBr, yAb, yBb,
                                 srcR2, dstR2, srcB2, dstB2, zeros)
    hu, t2Ar, t2Br, hi, t2Ab, t2Bb = _tcB(
        x_user, aAu, aBu, degU, W1_rb_self, b1_rb.reshape(1, D), W2_rates_neigh,
        x_item, aAi, aBi, degI, W1_rates_self, b1_rates.reshape(1, D), W2_rb_neigh)
    a2Ai, a2Bi, a2Au, a2Bu = _sc_agg(t2Ar, t2Br, t2Ab, t2Bb,
                                     srcR2, dstR2, srcB2, dstB2, zeros)
    hu2, hi2 = _tcC(hu, a2Au, a2Bu, degU, W2_rb_self, b2_rb.reshape(1, D),
                    hi, a2Ai, a2Bi, degI, W2_rates_self, b2_rates.reshape(1, D))
    return (hu2, hi2)


# speedup vs baseline: 17.3391x; 1.4124x over previous
import functools

import jax
import jax.numpy as jnp
from jax import lax
from jax.experimental import pallas as pl
from jax.experimental.pallas import tpu as pltpu
from jax.experimental.pallas import tpu_sc as plsc

N = 100000
D = 32
H = 16
E = 1600000
NS = 16
RPT = 784
ROWS = RPT * NS
EPAD = ROWS * 128
RB = 8
NBLK = RPT // RB
NACC = 100352
ZSTRIPE = NACC // NS
NP = NACC // 8
BP = NP // 8
GRID = NP // BP

_mesh = plsc.VectorSubcoreMesh(core_axis_name="c", subcore_axis_name="s")


@functools.partial(
    pl.kernel,
    out_type=[jax.ShapeDtypeStruct((NACC, H), jnp.float32)] * 4,
    mesh=_mesh,
    compiler_params=pltpu.CompilerParams(use_tc_tiling_on_sc=False),
    scratch_types=[
        pltpu.VMEM_SHARED((NACC, H), jnp.float32),
        pltpu.VMEM((RB, 128), jnp.int32),
        pltpu.VMEM((RB, 128), jnp.int32),
        pltpu.VMEM((RB * 128, H), jnp.float32),
        pltpu.SemaphoreType.DMA,
    ],
)
def _sc_agg(tA_r, tB_r, tA_b, tB_b, srcR, dstR, srcB, dstB, zeros,
            oAi, oBi, oAu, oBu, accum, sidx, didx, rows, sem):
    c = lax.axis_index("c")
    s = lax.axis_index("s")

    def do_rel(table, src2, dst2, out):
        pltpu.sync_copy(zeros, accum.at[pl.ds(s * ZSTRIPE, ZSTRIPE)])
        plsc.subcore_barrier()
        base = s * RPT

        def blk(b, carry):
            r0 = base + b * RB
            ics = [pltpu.async_copy(src2.at[pl.ds(r0, RB)], sidx, sem),
                   pltpu.async_copy(dst2.at[pl.ds(r0, RB)], didx, sem)]
            for cp in ics:
                cp.wait()
            gcs = [
                pltpu.async_copy(table.at[sidx.at[j]],
                                 rows.at[pl.ds(j * 128, 128)], sem)
                for j in range(RB)
            ]
            for cp in gcs:
                cp.wait()
            scs = [
                pltpu.async_copy(rows.at[pl.ds(j * 128, 128)],
                                 accum.at[didx.at[j]], sem, add=True)
                for j in range(RB)
            ]
            for cp in scs:
                cp.wait()
            return carry

        lax.fori_loop(0, NBLK, blk, 0)
        plsc.subcore_barrier()
        pltpu.sync_copy(accum.at[pl.ds(s * ZSTRIPE, ZSTRIPE)],
                        out.at[pl.ds(s * ZSTRIPE, ZSTRIPE)])
        plsc.subcore_barrier()

    @pl.when(c == 0)
    def _():
        do_rel(tA_r, srcR, dstR, oAi)
        do_rel(tA_b, srcB, dstB, oAu)

    @pl.when(c == 1)
    def _():
        do_rel(tB_r, srcR, dstR, oBi)
        do_rel(tB_b, srcB, dstB, oBu)


@functools.partial(
    pl.kernel,
    out_type=[jax.ShapeDtypeStruct((NACC, H), jnp.float32)] * 2,
    mesh=_mesh,
    compiler_params=pltpu.CompilerParams(use_tc_tiling_on_sc=False),
    scratch_types=[
        pltpu.VMEM_SHARED((NACC, H), jnp.float32),
        pltpu.VMEM((RB, 128), jnp.int32),
        pltpu.VMEM((128, H), jnp.float32),
        pltpu.SemaphoreType.DMA,
    ],
)
def _sc_deg(dstR, dstB, ones, zeros, oI, oU, accum, didx, ones_v, sem):
    c = lax.axis_index("c")
    s = lax.axis_index("s")
    pltpu.sync_copy(ones, ones_v)

    def do_rel(dst2, out):
        pltpu.sync_copy(zeros, accum.at[pl.ds(s * ZSTRIPE, ZSTRIPE)])
        plsc.subcore_barrier()
        base = s * RPT

        def blk(b, carry):
            r0 = base + b * RB
            pltpu.sync_copy(dst2.at[pl.ds(r0, RB)], didx)
            scs = [pltpu.async_copy(ones_v, accum.at[didx.at[j]], sem,
                                    add=True)
                   for j in range(RB)]
            for cp in scs:
                cp.wait()
            return carry

        lax.fori_loop(0, NBLK, blk, 0)
        plsc.subcore_barrier()
        pltpu.sync_copy(accum.at[pl.ds(s * ZSTRIPE, ZSTRIPE)],
                        out.at[pl.ds(s * ZSTRIPE, ZSTRIPE)])

    @pl.when(c == 0)
    def _():
        do_rel(dstR, oI)

    @pl.when(c == 1)
    def _():
        do_rel(dstB, oU)


def _rows(w):
    return pl.BlockSpec((BP, w), lambda i: (i, 0))


def _const(r, w):
    return pl.BlockSpec((r, w), lambda i: (0, 0))


_f32 = functools.partial(jnp.dot, preferred_element_type=jnp.float32)


def _tcA_body(xu, xi, wrA, wrB, wbA, wbB, yAr, yBr, yAb, yBb):
    yAr[...] = _f32(xu[...], wrA[...])
    yBr[...] = _f32(xu[...], wrB[...])
    yAb[...] = _f32(xi[...], wbA[...])
    yBb[...] = _f32(xi[...], wbB[...])


_tcA = pl.pallas_call(
    _tcA_body,
    grid=(GRID,),
    in_specs=[_rows(256), _rows(256)] + [_const(256, 128)] * 4,
    out_specs=[_rows(128)] * 4,
    out_shape=[jax.ShapeDtypeStruct((NP, 128), jnp.float32)] * 4,
)


def _mix(aA, aB, dg, eA, eB):
    inv = 1.0 / jnp.maximum(dg, 1.0)
    return _f32(aA * inv, eA) + _f32(aB * inv, eB)


def _tcB_body(xu, aAu, aBu, dgu, wsu, bu, w2rA, w2rB,
              xi, aAi, aBi, dgi, wsi, bi, w2bA, w2bB, eA, eB,
              hu, t2Ar, t2Br, hi, t2Ab, t2Bb):
    h_u = jnp.maximum(_f32(xu[...], wsu[...])
                      + _mix(aAu[...], aBu[...], dgu[...], eA[...], eB[...])
                      + bu[...], 0.0)
    hu[...] = h_u
    t2Ar[...] = _f32(h_u, w2rA[...])
    t2Br[...] = _f32(h_u, w2rB[...])
    h_i = jnp.maximum(_f32(xi[...], wsi[...])
                      + _mix(aAi[...], aBi[...], dgi[...], eA[...], eB[...])
                      + bi[...], 0.0)
    hi[...] = h_i
    t2Ab[...] = _f32(h_i, w2bA[...])
    t2Bb[...] = _f32(h_i, w2bB[...])


_tcB = pl.pallas_call(
    _tcB_body,
    grid=(GRID,),
    in_specs=[_rows(256), _rows(128), _rows(128), _rows(128),
              _const(256, 256), _const(1, 256), _const(256, 128), _const(256, 128),
              _rows(256), _rows(128), _rows(128), _rows(128),
              _const(256, 256), _const(1, 256), _const(256, 128), _const(256, 128),
              _const(128, 256), _const(128, 256)],
    out_specs=[_rows(256), _rows(128), _rows(128),
               _rows(256), _rows(128), _rows(128)],
    out_shape=[jax.ShapeDtypeStruct((NP, 256), jnp.float32),
               jax.ShapeDtypeStruct((NP, 128), jnp.float32),
               jax.ShapeDtypeStruct((NP, 128), jnp.float32),
               jax.ShapeDtypeStruct((NP, 256), jnp.float32),
               jax.ShapeDtypeStruct((NP, 128), jnp.float32),
               jax.ShapeDtypeStruct((NP, 128), jnp.float32)],
)


def _tcC_body(hu, aAu, aBu, dgu, wsu, bu,
              hi, aAi, aBi, dgi, wsi, bi, eA, eB,
              ou, oi):
    ou[...] = (_f32(hu[...], wsu[...])
               + _mix(aAu[...], aBu[...], dgu[...], eA[...], eB[...])
               + bu[...])
    oi[...] = (_f32(hi[...], wsi[...])
               + _mix(aAi[...], aBi[...], dgi[...], eA[...], eB[...])
               + bi[...])


_tcC = pl.pallas_call(
    _tcC_body,
    grid=(GRID,),
    in_specs=[_rows(256), _rows(128), _rows(128), _rows(128),
              _const(256, 256), _const(1, 256),
              _rows(256), _rows(128), _rows(128), _rows(128),
              _const(256, 256), _const(1, 256),
              _const(128, 256), _const(128, 256)],
    out_specs=[_rows(256), _rows(256)],
    out_shape=[jax.ShapeDtypeStruct((NP, 256), jnp.float32),
               jax.ShapeDtypeStruct((NP, 256), jnp.float32)],
)


def _pad_edges(src, dst):
    padn = EPAD - E
    pad_src = (jnp.arange(padn, dtype=jnp.int32) * 97) % N
    pad_dst = N + (jnp.arange(padn, dtype=jnp.int32) % (NACC - N))
    src2 = jnp.concatenate([src.astype(jnp.int32), pad_src]).reshape(ROWS, 128)
    dst2 = jnp.concatenate([dst.astype(jnp.int32), pad_dst]).reshape(ROWS, 128)
    return src2, dst2


def _pack(x):
    return jnp.pad(x, ((0, NACC - N), (0, 0))).reshape(NP, 8 * x.shape[1])


def _bd(W):
    return jnp.kron(jnp.eye(8, dtype=W.dtype), W)


def _as_table(p):
    return p.reshape(NACC, H)


def _as_packed(t):
    return t.reshape(NP, 128)


def kernel(x_user, x_item, edge_index_rates, edge_index_rated_by,
           W1_rates_self, W1_rates_neigh, W1_rb_self, W1_rb_neigh,
           W2_rates_self, W2_rates_neigh, W2_rb_self, W2_rb_neigh,
           b1_rates, b1_rb, b2_rates, b2_rb):
    srcR2, dstR2 = _pad_edges(edge_index_rates[0], edge_index_rates[1])
    srcB2, dstB2 = _pad_edges(edge_index_rated_by[0], edge_index_rated_by[1])
    zeros = jnp.zeros((ZSTRIPE, H), jnp.float32)
    ones = jnp.ones((128, H), jnp.float32)
    eye8 = jnp.eye(8, dtype=jnp.float32)
    selA = jnp.kron(eye8, jnp.eye(H, D, dtype=jnp.float32))
    selB = jnp.kron(eye8, jnp.eye(H, D, k=H, dtype=jnp.float32))

    xu_p = _pack(x_user)
    xi_p = _pack(x_item)

    degI, degU = _sc_deg(dstR2, dstB2, ones, zeros)
    degI_p, degU_p = _as_packed(degI), _as_packed(degU)

    yAr, yBr, yAb, yBb = _tcA(xu_p, xi_p,
                              _bd(W1_rates_neigh[:, :H]), _bd(W1_rates_neigh[:, H:]),
                              _bd(W1_rb_neigh[:, :H]), _bd(W1_rb_neigh[:, H:]))
    aAi, aBi, aAu, aBu = _sc_agg(_as_table(yAr), _as_table(yBr),
                                 _as_table(yAb), _as_table(yBb),
                                 srcR2, dstR2, srcB2, dstB2, zeros)
    hu, t2Ar, t2Br, hi, t2Ab, t2Bb = _tcB(
        xu_p, _as_packed(aAu), _as_packed(aBu), degU_p,
        _bd(W1_rb_self), jnp.tile(b1_rb, 8).reshape(1, 256),
        _bd(W2_rates_neigh[:, :H]), _bd(W2_rates_neigh[:, H:]),
        xi_p, _as_packed(aAi), _as_packed(aBi), degI_p,
        _bd(W1_rates_self), jnp.tile(b1_rates, 8).reshape(1, 256),
        _bd(W2_rb_neigh[:, :H]), _bd(W2_rb_neigh[:, H:]),
        selA, selB)
    a2Ai, a2Bi, a2Au, a2Bu = _sc_agg(_as_table(t2Ar), _as_table(t2Br),
                                     _as_table(t2Ab), _as_table(t2Bb),
                                     srcR2, dstR2, srcB2, dstB2, zeros)
    hu2, hi2 = _tcC(hu, _as_packed(a2Au), _as_packed(a2Bu), degU_p,
                    _bd(W2_rb_self), jnp.tile(b2_rb, 8).reshape(1, 256),
                    hi, _as_packed(a2Ai), _as_packed(a2Bi), degI_p,
                    _bd(W2_rates_self), jnp.tile(b2_rates, 8).reshape(1, 256),
                    selA, selB)
    out_u = hu2.reshape(NACC, D)[:N]
    out_i = hi2.reshape(NACC, D)[:N]
    return (out_u, out_i)
